# baseline (speedup 1.0000x reference)
import functools

import jax
import jax.numpy as jnp
from jax import lax
from jax.experimental import pallas as pl
from jax.experimental.pallas import tpu as pltpu
from jax.experimental.pallas import tpu_sc as plsc

_BLK = 256
_CHUNK = 64


def _dotT(a, b):
    return lax.dot_general(a, b, (((1,), (1,)), ((), ())),
                           preferred_element_type=jnp.float32)


def _router_body(x_ref, gw_ref, gb_ref, pos_ref, w_ref, blk_ref, nblk_ref):
    x = x_ref[...]
    T = x.shape[0]
    E = gw_ref.shape[0]
    B = _BLK
    logits = _dotT(x, gw_ref[...])
    s = jax.nn.sigmoid(logits)
    sc = s + gb_ref[...]
    lane = lax.broadcasted_iota(jnp.int32, sc.shape, 1)
    m1 = jnp.max(sc, axis=1, keepdims=True)
    i1 = jnp.min(jnp.where(sc == m1, lane, E), axis=1, keepdims=True)
    first1 = lane == i1
    scm = jnp.where(first1, -jnp.inf, sc)
    m2 = jnp.max(scm, axis=1, keepdims=True)
    i2 = jnp.min(jnp.where(scm == m2, lane, E), axis=1, keepdims=True)
    first2 = lane == i2
    w1 = jnp.sum(jnp.where(first1, s, 0.0), axis=1, keepdims=True)
    w2 = jnp.sum(jnp.where(first2, s, 0.0), axis=1, keepdims=True)
    wsum = w1 + w2 + 1e-20
    w_ref[...] = jnp.concatenate([w1, w2], axis=1) / wsum

    I = jnp.concatenate([first1.astype(jnp.float32),
                         first2.astype(jnp.float32)], axis=0)
    A = 2 * T
    C = 256
    r = lax.broadcasted_iota(jnp.int32, (C, C), 0)
    c = lax.broadcasted_iota(jnp.int32, (C, C), 1)
    tril = (c <= r).astype(jnp.float32)
    off = jnp.zeros((1, E), jnp.float32)
    ranks = []
    for ch in range(A // C):
        blk = lax.slice(I, (ch * C, 0), ((ch + 1) * C, E))
        pre = lax.dot_general(tril, blk, (((1,), (0,)), ((), ())),
                              preferred_element_type=jnp.float32)
        ranks.append(pre + off)
        off = off + lax.slice(pre, (C - 1, 0), (C, E))
    rank = jnp.concatenate(ranks, axis=0)
    cnt = off
    pad = jnp.floor((cnt + (B - 1)) / B) * B
    er = lax.broadcasted_iota(jnp.int32, (E, E), 0)
    ec = lax.broadcasted_iota(jnp.int32, (E, E), 1)
    sut = (er < ec).astype(jnp.float32)
    base = lax.dot_general(pad, sut, (((1,), (0,)), ((), ())),
                           preferred_element_type=jnp.float32)
    pos = jnp.sum(I * (base + rank - 1.0), axis=1, keepdims=True)
    pos_ref[...] = pos.astype(jnp.int32)

    nb = jnp.sum(pad, axis=1, keepdims=True) / B
    nblk_ref[...] = nb.astype(jnp.int32)
    brow = lax.broadcasted_iota(jnp.int32, (32, 1), 0).astype(jnp.float32)
    base_blk = base / B
    blk_ref[...] = (jnp.sum((brow >= base_blk).astype(jnp.float32), axis=1,
                            keepdims=True) - 1.0).astype(jnp.int32)


def _router_plan(x, gate_w, gate_bias):
    T, H = x.shape
    E = gate_w.shape[0]
    A = 2 * T
    return pl.pallas_call(
        _router_body,
        grid=(1,),
        in_specs=[
            pl.BlockSpec((T, H), lambda i: (0, 0)),
            pl.BlockSpec((E, H), lambda i: (0, 0)),
            pl.BlockSpec((1, E), lambda i: (0, 0)),
        ],
        out_specs=[
            pl.BlockSpec((A, 1), lambda i: (0, 0)),
            pl.BlockSpec((T, 2), lambda i: (0, 0)),
            pl.BlockSpec((32, 1), lambda i: (0, 0)),
            pl.BlockSpec((1, 1), lambda i: (0, 0)),
        ],
        out_shape=[
            jax.ShapeDtypeStruct((A, 1), jnp.int32),
            jax.ShapeDtypeStruct((T, 2), jnp.float32),
            jax.ShapeDtypeStruct((32, 1), jnp.int32),
            jax.ShapeDtypeStruct((1, 1), jnp.int32),
        ],
    )(x, gate_w, gate_bias.reshape(1, E))


def _sc_permute(src, pos3, P, gather_src):
    H = src.shape[1]
    NW, NCH, CH = pos3.shape
    A = NW * NCH * CH
    T = A // 2
    CPT = A // NW
    mesh = plsc.VectorSubcoreMesh(core_axis_name="c", subcore_axis_name="s")
    info = plsc.get_sparse_core_info()
    NC = info.num_cores

    @functools.partial(
        pl.kernel, mesh=mesh,
        out_type=jax.ShapeDtypeStruct((P if not gather_src else A, H),
                                      jnp.float32),
        scratch_types=[
            pltpu.VMEM((NCH, CH), jnp.int32),
            pltpu.VMEM((NCH, CH), jnp.int32),
            pltpu.VMEM((CH, H), jnp.float32),
            pltpu.SemaphoreType.DMA,
        ],
    )
    def k(src_hbm, pos_hbm, out_hbm, pos_v, tok_v, rows_v, sem):
        wid = lax.axis_index("s") * NC + lax.axis_index("c")
        base = wid * CPT
        pltpu.sync_copy(pos_hbm.at[wid], pos_v)
        iota = lax.broadcasted_iota(jnp.int32, (16,), 0)
        for j in range(NCH):
            for q in range(CH // 16):
                a = base + j * CH + q * 16 + iota
                tok_v[j, pl.ds(q * 16, 16)] = lax.rem(a, T)
        for j in range(NCH):
            if gather_src:
                pltpu.async_copy(src_hbm.at[pos_v.at[j]], rows_v, sem).wait()
                pltpu.sync_copy(
                    rows_v, out_hbm.at[pl.ds(base + j * CH, CH)])
            else:
                pltpu.async_copy(src_hbm.at[tok_v.at[j]], rows_v, sem).wait()
                pltpu.async_copy(rows_v, out_hbm.at[pos_v.at[j]], sem).wait()

    return k(src, pos3)


def _grouped_body(blk_ref, nblk_ref, xs_ref, wg_ref, wu_ref, wd_ref, y_ref):
    b = pl.program_id(0)

    @pl.when(b < nblk_ref[0])
    def _():
        xs = xs_ref[...]
        g = _dotT(xs, wg_ref[0])
        u = _dotT(xs, wu_ref[0])
        h = g * jax.nn.sigmoid(g) * u
        y_ref[...] = _dotT(h, wd_ref[0])


def _grouped_matmul(x_sorted, blk_exp, nblk, w_gate, w_up, w_down, NB):
    P, H = x_sorted.shape
    E, F, _ = w_gate.shape

    def live(b, blk, nb):
        return jnp.minimum(b, nb[0] - 1)

    grid_spec = pltpu.PrefetchScalarGridSpec(
        num_scalar_prefetch=2,
        grid=(NB,),
        in_specs=[
            pl.BlockSpec((_BLK, H), lambda b, blk, nb: (live(b, blk, nb), 0)),
            pl.BlockSpec((1, F, H),
                         lambda b, blk, nb: (blk[live(b, blk, nb)], 0, 0)),
            pl.BlockSpec((1, F, H),
                         lambda b, blk, nb: (blk[live(b, blk, nb)], 0, 0)),
            pl.BlockSpec((1, H, F),
                         lambda b, blk, nb: (blk[live(b, blk, nb)], 0, 0)),
        ],
        out_specs=pl.BlockSpec((_BLK, H),
                               lambda b, blk, nb: (live(b, blk, nb), 0)),
    )
    return pl.pallas_call(
        _grouped_body,
        grid_spec=grid_spec,
        out_shape=jax.ShapeDtypeStruct((P, H), jnp.float32),
        compiler_params=pltpu.CompilerParams(
            dimension_semantics=("arbitrary",)),
    )(blk_exp, nblk, x_sorted, w_gate, w_up, w_down)


def _combine_body(p0_ref, p1_ref, w_ref, x_ref, wsg_ref, wsu_ref, wsd_ref,
                  out_ref):
    x = x_ref[...]
    gs = _dotT(x, wsg_ref[...])
    us = _dotT(x, wsu_ref[...])
    hs = gs * jax.nn.sigmoid(gs) * us
    shared = _dotT(hs, wsd_ref[...])
    w0 = w_ref[:, 0:1]
    w1 = w_ref[:, 1:2]
    out_ref[...] = shared + w0 * p0_ref[...] + w1 * p1_ref[...]


def _combine(pairs, wpair, x, ws_gate, ws_up, ws_down):
    T, H = x.shape
    SF = ws_gate.shape[0]
    TBS = min(1024, T)
    NT = T // TBS
    return pl.pallas_call(
        _combine_body,
        grid=(NT,),
        in_specs=[
            pl.BlockSpec((TBS, H), lambda t: (t, 0)),
            pl.BlockSpec((TBS, H), lambda t: (t + NT, 0)),
            pl.BlockSpec((TBS, 2), lambda t: (t, 0)),
            pl.BlockSpec((TBS, H), lambda t: (t, 0)),
            pl.BlockSpec((SF, H), lambda t: (0, 0)),
            pl.BlockSpec((SF, H), lambda t: (0, 0)),
            pl.BlockSpec((H, SF), lambda t: (0, 0)),
        ],
        out_specs=pl.BlockSpec((TBS, H), lambda t: (t, 0)),
        out_shape=jax.ShapeDtypeStruct((T, H), jnp.float32),
        compiler_params=pltpu.CompilerParams(
            dimension_semantics=("arbitrary",)),
    )(pairs, pairs, wpair, x, ws_gate, ws_up, ws_down)


def kernel(hidden_states, gate_w, gate_bias, w_gate, w_up, w_down,
           ws_gate, ws_up, ws_down):
    T, H = hidden_states.shape
    E = gate_w.shape[0]
    A = 2 * T
    NB = A // _BLK + E
    P = NB * _BLK

    pos, wpair, blk_exp, nblk = _router_plan(hidden_states, gate_w, gate_bias)
    CPT = A // 32
    CH = min(_CHUNK, CPT)
    pos3 = pos.reshape(32, CPT // CH, CH)
    x_sorted = _sc_permute(hidden_states, pos3, P, gather_src=False)
    y_sorted = _grouped_matmul(x_sorted, blk_exp.reshape(32), nblk.reshape(1),
                               w_gate, w_up, w_down, NB)
    pairs = _sc_permute(y_sorted, pos3, P, gather_src=True)
    return _combine(pairs, wpair, hidden_states, ws_gate, ws_up, ws_down)
